# Initial kernel scaffold; baseline (speedup 1.0000x reference)
import jax
import jax.numpy as jnp
from jax.experimental import pallas as pl


def kernel(x, edge_index, batch, Wrel0, brel0, Wroot0, Wrel1, brel1, Wroot1, Wrel2, brel2, Wroot2, Wrel3, brel3, Wroot3, Wrel4, brel4, Wroot4, W1, b1, gamma, beta, W2, b2):
    raise NotImplementedError("write your pallas kernel here")


# speedup vs baseline: 4.6315x; 4.6315x over previous
import functools

import jax
import jax.numpy as jnp
from jax import lax
from jax.experimental import pallas as pl
from jax.experimental.pallas import tpu as pltpu
from jax.experimental.pallas import tpu_sc as plsc

_N = 10000
_E = 320000
_D = 128
_G = 128
_NC = 2
_NS = 16
_NW = _NC * _NS
_NP = 10240
_RPW = 320
_TRASH = _NS * _RPW
_ACC_R = _NS * _RPW + 8
_SB = 8192
_STG = _SB + 160
_BATCH = 128
_EPAD = _E + _STG
_BR = 1000


def _agg_body(h_hbm, src_hbm, dst_hbm, offs_hbm, zeros_hbm, out,
              acc, vs, vd, ba_s, ba_d, bb_s, bb_d, rows_a, rows_b, offs_v,
              sem_a, sem_b, sem_s, sem_d):
  cid = lax.axis_index("c")
  sid = lax.axis_index("s")
  wid = cid * _NS + sid
  rel0 = sid * _RPW

  pltpu.sync_copy(zeros_hbm.at[pl.ds(0, _RPW)], acc.at[pl.ds(rel0, _RPW)])

  @pl.when(sid == 0)
  def _():
    pltpu.sync_copy(zeros_hbm.at[pl.ds(0, 8)], acc.at[pl.ds(_TRASH, 8)])

  pltpu.sync_copy(offs_hbm, offs_v)
  off0 = offs_v[pl.ds(wid, 16)][0]
  off1 = offs_v[pl.ds(wid + 1, 16)][0]
  abase = pl.multiple_of(off0 - jnp.bitwise_and(off0, 7), 8)
  nblk = jnp.maximum((off1 - abase + _SB - 1) >> 13, 0)

  iota16 = lax.iota(jnp.int32, 16)

  def _build(bat_s, bat_d, p0, bstart, hi_i):
    for g in range(8):
      loc = p0 - bstart + g * 16
      s = vs[pl.ds(loc, 16)]
      d = vd[pl.ds(loc, 16)]
      p = (p0 + g * 16) + iota16
      m = (p >= off0) & (p < hi_i)
      bat_s[pl.ds(g * 16, 16)] = jnp.where(m, s, 0)
      bat_d[pl.ds(g * 16, 16)] = jnp.where(m, d, _TRASH)

  def block(i, carry):
    bstart = pl.multiple_of(abase + i * _SB, 8)
    pltpu.sync_copy(src_hbm.at[pl.ds(bstart, _SB)], vs.at[pl.ds(0, _SB)])
    pltpu.sync_copy(dst_hbm.at[pl.ds(bstart, _SB)], vd.at[pl.ds(0, _SB)])
    lo_i = jnp.maximum(off0, bstart)
    hi_i = jnp.minimum(off1, bstart + _SB)
    nbat = jnp.maximum((hi_i - lo_i + _BATCH - 1) >> 7, 0)

    def pair(k, c2):
      b0 = k * 2
      _build(ba_s, ba_d, lo_i + b0 * _BATCH, bstart, hi_i)
      pltpu.async_copy(h_hbm.at[ba_s], rows_a, sem_a)

      @pl.when(b0 + 1 < nbat)
      def _():
        _build(bb_s, bb_d, lo_i + (b0 + 1) * _BATCH, bstart, hi_i)
        pltpu.async_copy(h_hbm.at[bb_s], rows_b, sem_b)

      pltpu.make_async_copy(h_hbm.at[ba_s], rows_a, sem_a).wait()
      pltpu.sync_copy(rows_a, acc.at[ba_d], add=True)

      @pl.when(b0 + 1 < nbat)
      def _():
        pltpu.make_async_copy(h_hbm.at[bb_s], rows_b, sem_b).wait()
        pltpu.sync_copy(rows_b, acc.at[bb_d], add=True)

      return c2

    lax.fori_loop(0, (nbat + 1) >> 1, pair, 0)
    return carry

  lax.fori_loop(0, nblk, block, 0)

  pltpu.sync_copy(acc.at[pl.ds(rel0, _RPW)], out.at[pl.ds(wid * _RPW, _RPW)])


@functools.cache
def _make_agg():
  return pl.kernel(
      _agg_body,
      out_type=jax.ShapeDtypeStruct((_NP, _D), jnp.float32),
      mesh=plsc.VectorSubcoreMesh(core_axis_name="c", subcore_axis_name="s",
                                  num_cores=_NC, num_subcores=_NS),
      scratch_types=(
          pltpu.VMEM_SHARED((_ACC_R, _D), jnp.float32),
          pltpu.VMEM((_STG,), jnp.int32),
          pltpu.VMEM((_STG,), jnp.int32),
          pltpu.VMEM((_BATCH,), jnp.int32),
          pltpu.VMEM((_BATCH,), jnp.int32),
          pltpu.VMEM((_BATCH,), jnp.int32),
          pltpu.VMEM((_BATCH,), jnp.int32),
          pltpu.VMEM((_BATCH, _D), jnp.float32),
          pltpu.VMEM((_BATCH, _D), jnp.float32),
          pltpu.VMEM((48,), jnp.int32),
          pltpu.SemaphoreType.DMA,
          pltpu.SemaphoreType.DMA,
          pltpu.SemaphoreType.DMA,
          pltpu.SemaphoreType.DMA,
      ),
  )


def _mm(a, b):
  return lax.dot_general(a, b, (((1,), (0,)), ((), ())),
                         preferred_element_type=jnp.float32)


def _dense_body(a, h, wrt, wrot, br, o):
  acc = _mm(a[...], wrt[...]) + _mm(h[...], wrot[...])
  o[...] = jnp.maximum(acc + br[...], 0.0)


def _dense_layer(a, h, WrT, WroT, br):
  return pl.pallas_call(
      _dense_body,
      grid=(_N // _BR,),
      in_specs=[
          pl.BlockSpec((_BR, _D), lambda i: (i, 0)),
          pl.BlockSpec((_BR, _D), lambda i: (i, 0)),
          pl.BlockSpec((_D, _D), lambda i: (0, 0)),
          pl.BlockSpec((_D, _D), lambda i: (0, 0)),
          pl.BlockSpec((1, _D), lambda i: (0, 0)),
      ],
      out_specs=pl.BlockSpec((_BR, _D), lambda i: (i, 0)),
      out_shape=jax.ShapeDtypeStruct((_N, _D), jnp.float32),
  )(a, h, WrT, WroT, br)


def _final_body(a, h, wrt, wrot, br, bref, w1t, b1, gm, be, w2t, b2,
                o, pooled):
  i = pl.program_id(0)
  h5 = _mm(a[...], wrt[...]) + _mm(h[...], wrot[...])
  h5 = jnp.maximum(h5 + br[...], 0.0)
  seg = bref[0, 0, :]
  onehot_t = (lax.broadcasted_iota(jnp.int32, (_G, _BR), 0)
              == seg[None, :]).astype(jnp.float32)
  part = lax.dot_general(onehot_t, h5, (((1,), (0,)), ((), ())),
                         preferred_element_type=jnp.float32,
                         precision=lax.Precision.HIGHEST)

  @pl.when(i == 0)
  def _():
    pooled[...] = part

  @pl.when(i > 0)
  def _():
    pooled[...] += part

  @pl.when(i == _N // _BR - 1)
  def _():
    z = _mm(pooled[...], w1t[...]) + b1[...]
    mean = jnp.mean(z, axis=0, keepdims=True)
    var = jnp.mean((z - mean) ** 2, axis=0, keepdims=True)
    zn = (z - mean) * lax.rsqrt(var + 1e-5) * gm[...] + be[...]
    zn = jnp.maximum(zn, 0.0)
    o[...] = _mm(zn, w2t[...]) + b2[...]


def _final_layer(a, h, WrT, WroT, br, batch3, W1T, b1, gm, be, W2T, b2):
  row = lambda i: (i, 0)
  full = lambda i: (0, 0)
  return pl.pallas_call(
      _final_body,
      grid=(_N // _BR,),
      in_specs=[
          pl.BlockSpec((_BR, _D), row),
          pl.BlockSpec((_BR, _D), row),
          pl.BlockSpec((_D, _D), full),
          pl.BlockSpec((_D, _D), full),
          pl.BlockSpec((1, _D), full),
          pl.BlockSpec((1, 1, _BR), lambda i: (i, 0, 0)),
          pl.BlockSpec((_D, _D), full),
          pl.BlockSpec((1, _D), full),
          pl.BlockSpec((1, _D), full),
          pl.BlockSpec((1, _D), full),
          pl.BlockSpec((_D, 64), full),
          pl.BlockSpec((1, 64), full),
      ],
      out_specs=pl.BlockSpec((_G, 64), full),
      out_shape=jax.ShapeDtypeStruct((_G, 64), jnp.float32),
      scratch_shapes=[pltpu.VMEM((_G, _D), jnp.float32)],
  )(a, h, WrT, WroT, br, batch3, W1T, b1, gm, be, W2T, b2)


def kernel(x, edge_index, batch, Wrel0, brel0, Wroot0, Wrel1, brel1, Wroot1,
           Wrel2, brel2, Wroot2, Wrel3, brel3, Wroot3, Wrel4, brel4, Wroot4,
           W1, b1, gamma, beta, W2, b2):
  src = edge_index[0].astype(jnp.int32)
  dst = edge_index[1].astype(jnp.int32)

  bucket = dst // _RPW
  perm = jnp.argsort(bucket, stable=True)
  src_s = jnp.concatenate(
      [src[perm], jnp.zeros((_EPAD - _E,), jnp.int32)])
  dst_p = dst[perm]
  dst_s = jnp.concatenate(
      [dst_p - (dst_p // _TRASH) * _TRASH,
       jnp.full((_EPAD - _E,), _TRASH, jnp.int32)])
  counts = jnp.sum(bucket[None, :] == jnp.arange(_NW, dtype=jnp.int32)[:, None],
                   axis=1, dtype=jnp.int32)
  offs = jnp.concatenate(
      [jnp.zeros((1,), jnp.int32), jnp.cumsum(counts, dtype=jnp.int32),
       jnp.full((48 - _NW - 1,), _E, jnp.int32)])

  zeros = jnp.zeros((_RPW, _D), jnp.float32)
  batch3 = batch.astype(jnp.int32).reshape(_N // _BR, 1, _BR)

  agg = _make_agg()
  convs = ((Wrel0, brel0, Wroot0), (Wrel1, brel1, Wroot1),
           (Wrel2, brel2, Wroot2), (Wrel3, brel3, Wroot3))
  h = x
  for Wr, br, Wro in convs:
    a = agg(h, src_s, dst_s, offs, zeros)
    h = _dense_layer(a, h, Wr.T, Wro.T, br.reshape(1, _D))

  a = agg(h, src_s, dst_s, offs, zeros)
  return _final_layer(a, h, Wrel4.T, Wroot4.T, brel4.reshape(1, _D),
                      batch3, W1.T, b1.reshape(1, _D), gamma.reshape(1, _D),
                      beta.reshape(1, _D), W2.T, b2.reshape(1, 64))
